# baseline (speedup 1.0000x reference)
import functools

import jax
import jax.numpy as jnp
from jax import lax
from jax.experimental import pallas as pl
from jax.experimental.pallas import tpu as pltpu
from jax.experimental.pallas import tpu_sc as plsc

H = 1024
F = 2048
E = 16
S = 2048
K = 2
P = K * S
TILE = 128
NT = 48
PAD = NT * TILE

NC, NS = 2, 16
NW = NC * NS
PPW = P // NW
CH = 32
NCH = PPW // CH


def _router_body(x_ref, wr_ref, inv_ref, w_ref, rowb_ref, ntl_ref):
    x = x_ref[...]
    logits = jnp.dot(x, wr_ref[...], preferred_element_type=jnp.float32)
    m = jnp.max(logits, axis=-1, keepdims=True)
    ex = jnp.exp(logits - m)
    probs = ex / jnp.sum(ex, axis=-1, keepdims=True)
    ids = lax.broadcasted_iota(jnp.int32, probs.shape, 1)
    m1 = jnp.max(probs, axis=-1, keepdims=True)
    i1 = jnp.min(jnp.where(probs == m1, ids, E), axis=-1, keepdims=True)
    masked = jnp.where(ids == i1, -1.0, probs)
    m2 = jnp.max(masked, axis=-1, keepdims=True)
    i2 = jnp.min(jnp.where(masked == m2, ids, E), axis=-1, keepdims=True)
    denom = m1 + m2
    w_ref[...] = jnp.concatenate([m1 / denom, m2 / denom], axis=1)

    e_all = jnp.concatenate([i1, i2], axis=0)
    onehot = (e_all == lax.broadcasted_iota(jnp.int32, (P, E), 1)
              ).astype(jnp.float32)
    c = onehot
    d = 1
    while d < P:
        c = c + jnp.concatenate(
            [jnp.zeros((d, E), jnp.float32), c[:-d, :]], axis=0)
        d *= 2
    counts = c[P - 1:P, :]
    tiles = jnp.floor((counts + (TILE - 1)) * (1.0 / TILE))
    incl = tiles
    d = 1
    while d < E:
        incl = incl + jnp.concatenate(
            [jnp.zeros((1, d), jnp.float32), incl[:, :-d]], axis=1)
        d *= 2
    tb_excl = incl - tiles
    base = tb_excl * TILE
    slot = jnp.sum((base + c - 1.0) * onehot, axis=1, keepdims=True)
    inv_ref[...] = slot.astype(jnp.int32)
    rowb_ref[...] = base.astype(jnp.int32)
    ntl_ref[...] = tiles.astype(jnp.int32)


def _gmm_body(rowb_ref, ntl_ref, xs_ref, w1_ref, v1_ref, w2_ref, ys_ref,
              xbuf, ybuf, yacc, insem, outsem):
    e = pl.program_id(0)
    fb = pl.program_id(1)
    base = rowb_ref[e]
    n = ntl_ref[e]
    w1 = w1_ref[0]
    v1 = v1_ref[0]
    w2 = w2_ref[0]

    def body(t, _):
        r0 = pl.multiple_of(base + t * TILE, TILE)
        rl = pl.multiple_of(t * TILE, TILE)
        cin = pltpu.make_async_copy(xs_ref.at[pl.ds(r0, TILE)], xbuf, insem)
        cin.start()
        cin.wait()
        xt = xbuf[...]
        a = jnp.dot(xt, w1, preferred_element_type=jnp.float32)
        bb = jnp.dot(xt, v1, preferred_element_type=jnp.float32)
        h = a * jax.nn.sigmoid(a) * bb
        y = jnp.dot(h, w2, preferred_element_type=jnp.float32)

        @pl.when(fb == 0)
        def _():
            yacc[pl.ds(rl, TILE), :] = y

        @pl.when(fb == 1)
        def _():
            ybuf[...] = yacc[pl.ds(rl, TILE), :] + y
            cout = pltpu.make_async_copy(
                ybuf, ys_ref.at[pl.ds(r0, TILE)], outsem)
            cout.start()
            cout.wait()

        return 0

    lax.fori_loop(0, n, body, 0)


def _combine_body(w_ref, y0_ref, y1_ref, out_ref):
    w0 = w_ref[:, 0:1]
    w1 = w_ref[:, 1:2]
    out_ref[...] = w0 * y0_ref[...] + w1 * y1_ref[...]


def _dispatch_x_body(x_hbm, inv_hbm, xs_hbm, idx_v, rows_v, sem):
    wid = lax.axis_index("s") * NC + lax.axis_index("c")
    pltpu.sync_copy(inv_hbm.at[pl.ds(wid * NCH, NCH)], idx_v)
    for c in range(NCH):
        t0 = lax.rem(wid * PPW + c * CH, S)
        pltpu.sync_copy(x_hbm.at[pl.ds(t0, CH)], rows_v)
        pltpu.async_copy(rows_v, xs_hbm.at[idx_v.at[c]], sem).wait()


def _gather_y_body(ys_hbm, inv_hbm, ysg_hbm, idx_v, rows_v, sem):
    wid = lax.axis_index("s") * NC + lax.axis_index("c")
    pltpu.sync_copy(inv_hbm.at[pl.ds(wid * NCH, NCH)], idx_v)
    for c in range(NCH):
        pltpu.async_copy(ys_hbm.at[idx_v.at[c]], rows_v, sem).wait()
        pltpu.sync_copy(rows_v, ysg_hbm.at[pl.ds(wid * PPW + c * CH, CH)])


def _sc_call(body, out_rows):
    mesh = plsc.VectorSubcoreMesh(
        core_axis_name="c", subcore_axis_name="s", num_cores=NC)
    return pl.kernel(
        body,
        mesh=mesh,
        out_type=jax.ShapeDtypeStruct((out_rows, H), jnp.float32),
        scratch_types=[
            pltpu.VMEM((NCH, CH), jnp.int32),
            pltpu.VMEM((CH, H), jnp.float32),
            pltpu.SemaphoreType.DMA,
        ],
    )


@jax.jit
def kernel(x, Wr, W1, V1, W2):
    b, s, hd = x.shape
    xf = x.reshape(s, hd)

    inv, wcomb, rowb, ntl = pl.pallas_call(
        _router_body,
        out_shape=(
            jax.ShapeDtypeStruct((P, 1), jnp.int32),
            jax.ShapeDtypeStruct((S, K), jnp.float32),
            jax.ShapeDtypeStruct((1, E), jnp.int32),
            jax.ShapeDtypeStruct((1, E), jnp.int32),
        ),
    )(xf, Wr)

    inv_chunks = inv.reshape(NW * NCH, CH)

    xs = _sc_call(_dispatch_x_body, PAD)(xf, inv_chunks)

    ys = pl.pallas_call(
        _gmm_body,
        grid_spec=pltpu.PrefetchScalarGridSpec(
            num_scalar_prefetch=2,
            grid=(E, 2),
            in_specs=[
                pl.BlockSpec(memory_space=pltpu.MemorySpace.HBM),
                pl.BlockSpec((1, H, F // 2), lambda e, fb, rb, nt: (e, 0, fb)),
                pl.BlockSpec((1, H, F // 2), lambda e, fb, rb, nt: (e, 0, fb)),
                pl.BlockSpec((1, F // 2, H), lambda e, fb, rb, nt: (e, fb, 0)),
            ],
            out_specs=pl.BlockSpec(memory_space=pltpu.MemorySpace.HBM),
            scratch_shapes=[
                pltpu.VMEM((TILE, H), jnp.float32),
                pltpu.VMEM((TILE, H), jnp.float32),
                pltpu.VMEM((P, H), jnp.float32),
                pltpu.SemaphoreType.DMA,
                pltpu.SemaphoreType.DMA,
            ],
        ),
        out_shape=jax.ShapeDtypeStruct((PAD, H), jnp.float32),
        compiler_params=pltpu.CompilerParams(
            vmem_limit_bytes=100 * 1024 * 1024),
    )(rowb.reshape(E), ntl.reshape(E), xs, W1, V1, W2)

    ysg = _sc_call(_gather_y_body, P)(ys, inv_chunks)

    out = pl.pallas_call(
        _combine_body,
        grid=(1,),
        in_specs=[
            pl.BlockSpec((S, K), lambda i: (0, 0)),
            pl.BlockSpec((S, H), lambda i: (0, 0)),
            pl.BlockSpec((S, H), lambda i: (1, 0)),
        ],
        out_specs=pl.BlockSpec((S, H), lambda i: (0, 0)),
        out_shape=jax.ShapeDtypeStruct((S, H), jnp.float32),
    )(wcomb, ysg, ysg)

    return out.reshape(b, s, hd)


# speedup vs baseline: 1.1815x; 1.1815x over previous
import functools

import jax
import jax.numpy as jnp
from jax import lax
from jax.experimental import pallas as pl
from jax.experimental.pallas import tpu as pltpu
from jax.experimental.pallas import tpu_sc as plsc

H = 1024
F = 2048
E = 16
S = 2048
K = 2
P = K * S
TILE = 128
NT = 48
PAD = NT * TILE

NC, NS = 2, 16
NW = NC * NS
PPW = P // NW
CH = 32
NCH = PPW // CH


def _router_body(x_ref, wr_ref, inv_ref, w_ref, rowb_ref, ntl_ref):
    x = x_ref[...]
    logits = jnp.dot(x, wr_ref[...], preferred_element_type=jnp.float32)
    m = jnp.max(logits, axis=-1, keepdims=True)
    ex = jnp.exp(logits - m)
    probs = ex / jnp.sum(ex, axis=-1, keepdims=True)
    ids = lax.broadcasted_iota(jnp.int32, probs.shape, 1)
    m1 = jnp.max(probs, axis=-1, keepdims=True)
    i1 = jnp.min(jnp.where(probs == m1, ids, E), axis=-1, keepdims=True)
    masked = jnp.where(ids == i1, -1.0, probs)
    m2 = jnp.max(masked, axis=-1, keepdims=True)
    i2 = jnp.min(jnp.where(masked == m2, ids, E), axis=-1, keepdims=True)
    denom = m1 + m2
    w_ref[...] = jnp.concatenate([m1 / denom, m2 / denom], axis=1)

    e_all = jnp.concatenate([i1, i2], axis=0)
    onehot = (e_all == lax.broadcasted_iota(jnp.int32, (P, E), 1)
              ).astype(jnp.float32)
    c = onehot
    d = 1
    while d < P:
        c = c + jnp.concatenate(
            [jnp.zeros((d, E), jnp.float32), c[:-d, :]], axis=0)
        d *= 2
    counts = c[P - 1:P, :]
    tiles = jnp.floor((counts + (TILE - 1)) * (1.0 / TILE))
    incl = tiles
    d = 1
    while d < E:
        incl = incl + jnp.concatenate(
            [jnp.zeros((1, d), jnp.float32), incl[:, :-d]], axis=1)
        d *= 2
    tb_excl = incl - tiles
    base = tb_excl * TILE
    slot = jnp.sum((base + c - 1.0) * onehot, axis=1, keepdims=True)
    inv_ref[...] = slot.astype(jnp.int32)
    rowb_ref[...] = base.astype(jnp.int32)
    ntl_ref[...] = tiles.astype(jnp.int32)


def _gmm_body(rowb_ref, ntl_ref, xs_ref, w1_ref, v1_ref, w2_ref, ys_ref,
              xbuf0, xbuf1, ybuf0, ybuf1, yacc,
              isem0, isem1, osem0, osem1):
    e = pl.program_id(0)
    fb = pl.program_id(1)
    base = rowb_ref[e]
    n = ntl_ref[e]
    w1 = w1_ref[0]
    v1 = v1_ref[0]
    w2 = w2_ref[0]

    def start_in(t, buf, sem):
        r0 = pl.multiple_of(base + t * TILE, TILE)
        pltpu.make_async_copy(xs_ref.at[pl.ds(r0, TILE)], buf, sem).start()

    @pl.when(n > 0)
    def _():
        start_in(0, xbuf0, isem0)

    def work(t, xbuf, isem, ybuf, osem):
        r0 = pl.multiple_of(base + t * TILE, TILE)
        rl = pl.multiple_of(t * TILE, TILE)
        pltpu.make_async_copy(xs_ref.at[pl.ds(r0, TILE)], xbuf, isem).wait()
        xt = xbuf[...]
        a = jnp.dot(xt, w1, preferred_element_type=jnp.float32)
        bb = jnp.dot(xt, v1, preferred_element_type=jnp.float32)
        h = a * jax.nn.sigmoid(a) * bb
        y = jnp.dot(h, w2, preferred_element_type=jnp.float32)

        @pl.when(fb == 0)
        def _():
            yacc[pl.ds(rl, TILE), :] = y

        @pl.when(fb == 1)
        def _():
            @pl.when(t >= 2)
            def _():
                pltpu.make_async_copy(
                    ybuf, ys_ref.at[pl.ds(r0, TILE)], osem).wait()

            ybuf[...] = yacc[pl.ds(rl, TILE), :] + y
            pltpu.make_async_copy(
                ybuf, ys_ref.at[pl.ds(r0, TILE)], osem).start()

    def body(t, _):
        par = lax.rem(t, 2)

        @pl.when(jnp.logical_and(par == 0, t + 1 < n))
        def _():
            start_in(t + 1, xbuf1, isem1)

        @pl.when(jnp.logical_and(par == 1, t + 1 < n))
        def _():
            start_in(t + 1, xbuf0, isem0)

        @pl.when(par == 0)
        def _():
            work(t, xbuf0, isem0, ybuf0, osem0)

        @pl.when(par == 1)
        def _():
            work(t, xbuf1, isem1, ybuf1, osem1)

        return 0

    lax.fori_loop(0, n, body, 0)

    @pl.when(jnp.logical_and(fb == 1, n >= 1))
    def _():
        pltpu.make_async_copy(ybuf0, ys_ref.at[pl.ds(0, TILE)], osem0).wait()

    @pl.when(jnp.logical_and(fb == 1, n >= 2))
    def _():
        pltpu.make_async_copy(ybuf1, ys_ref.at[pl.ds(0, TILE)], osem1).wait()


def _combine_body(w_ref, y0_ref, y1_ref, out_ref):
    w0 = w_ref[:, 0:1]
    w1 = w_ref[:, 1:2]
    out_ref[...] = w0 * y0_ref[...] + w1 * y1_ref[...]


def _dispatch_x_body(x_hbm, inv_hbm, xs_hbm, idx_v, rows_v, sem):
    wid = lax.axis_index("s") * NC + lax.axis_index("c")
    pltpu.sync_copy(inv_hbm.at[pl.ds(wid * NCH, NCH)], idx_v)
    for c in range(NCH):
        t0 = lax.rem(wid * PPW + c * CH, S)
        pltpu.sync_copy(x_hbm.at[pl.ds(t0, CH)], rows_v)
        pltpu.async_copy(rows_v, xs_hbm.at[idx_v.at[c]], sem).wait()


def _gather_y_body(ys_hbm, inv_hbm, ysg_hbm, idx_v, rows_v, sem):
    wid = lax.axis_index("s") * NC + lax.axis_index("c")
    pltpu.sync_copy(inv_hbm.at[pl.ds(wid * NCH, NCH)], idx_v)
    for c in range(NCH):
        pltpu.async_copy(ys_hbm.at[idx_v.at[c]], rows_v, sem).wait()
        pltpu.sync_copy(rows_v, ysg_hbm.at[pl.ds(wid * PPW + c * CH, CH)])


def _sc_call(body, out_rows):
    mesh = plsc.VectorSubcoreMesh(
        core_axis_name="c", subcore_axis_name="s", num_cores=NC)
    return pl.kernel(
        body,
        mesh=mesh,
        out_type=jax.ShapeDtypeStruct((out_rows, H), jnp.float32),
        scratch_types=[
            pltpu.VMEM((NCH, CH), jnp.int32),
            pltpu.VMEM((CH, H), jnp.float32),
            pltpu.SemaphoreType.DMA,
        ],
    )


@jax.jit
def kernel(x, Wr, W1, V1, W2):
    b, s, hd = x.shape
    xf = x.reshape(s, hd)

    inv, wcomb, rowb, ntl = pl.pallas_call(
        _router_body,
        out_shape=(
            jax.ShapeDtypeStruct((P, 1), jnp.int32),
            jax.ShapeDtypeStruct((S, K), jnp.float32),
            jax.ShapeDtypeStruct((1, E), jnp.int32),
            jax.ShapeDtypeStruct((1, E), jnp.int32),
        ),
    )(xf, Wr)

    inv_chunks = inv.reshape(NW * NCH, CH)

    xs = _sc_call(_dispatch_x_body, PAD)(xf, inv_chunks)

    ys = pl.pallas_call(
        _gmm_body,
        grid_spec=pltpu.PrefetchScalarGridSpec(
            num_scalar_prefetch=2,
            grid=(E, 2),
            in_specs=[
                pl.BlockSpec(memory_space=pltpu.MemorySpace.HBM),
                pl.BlockSpec((1, H, F // 2), lambda e, fb, rb, nt: (e, 0, fb)),
                pl.BlockSpec((1, H, F // 2), lambda e, fb, rb, nt: (e, 0, fb)),
                pl.BlockSpec((1, F // 2, H), lambda e, fb, rb, nt: (e, fb, 0)),
            ],
            out_specs=pl.BlockSpec(memory_space=pltpu.MemorySpace.HBM),
            scratch_shapes=[
                pltpu.VMEM((TILE, H), jnp.float32),
                pltpu.VMEM((TILE, H), jnp.float32),
                pltpu.VMEM((TILE, H), jnp.float32),
                pltpu.VMEM((TILE, H), jnp.float32),
                pltpu.VMEM((P, H), jnp.float32),
                pltpu.SemaphoreType.DMA,
                pltpu.SemaphoreType.DMA,
                pltpu.SemaphoreType.DMA,
                pltpu.SemaphoreType.DMA,
            ],
        ),
        out_shape=jax.ShapeDtypeStruct((PAD, H), jnp.float32),
        compiler_params=pltpu.CompilerParams(
            vmem_limit_bytes=100 * 1024 * 1024),
    )(rowb.reshape(E), ntl.reshape(E), xs, W1, V1, W2)

    ysg = _sc_call(_gather_y_body, P)(ys, inv_chunks)

    out = pl.pallas_call(
        _combine_body,
        grid=(1,),
        in_specs=[
            pl.BlockSpec((S, K), lambda i: (0, 0)),
            pl.BlockSpec((S, H), lambda i: (0, 0)),
            pl.BlockSpec((S, H), lambda i: (1, 0)),
        ],
        out_specs=pl.BlockSpec((S, H), lambda i: (0, 0)),
        out_shape=jax.ShapeDtypeStruct((S, H), jnp.float32),
    )(wcomb, ysg, ysg)

    return out.reshape(b, s, hd)


# speedup vs baseline: 1.7162x; 1.4525x over previous
import functools

import jax
import jax.numpy as jnp
from jax import lax
from jax.experimental import pallas as pl
from jax.experimental.pallas import tpu as pltpu
from jax.experimental.pallas import tpu_sc as plsc

H = 1024
F = 2048
E = 16
S = 2048
K = 2
P = K * S
TILE = 128
NT = 48
PAD = NT * TILE

NC, NS = 2, 16
NW = NC * NS
PPW = P // NW
CH = 32
NCH = PPW // CH


def _router_body(x_ref, wr_ref, inv_ref, w_ref, gidx_ref, ntot_ref):
    x = x_ref[...]
    logits = jnp.dot(x, wr_ref[...], preferred_element_type=jnp.float32)
    m = jnp.max(logits, axis=-1, keepdims=True)
    ex = jnp.exp(logits - m)
    probs = ex / jnp.sum(ex, axis=-1, keepdims=True)
    ids = lax.broadcasted_iota(jnp.int32, probs.shape, 1)
    m1 = jnp.max(probs, axis=-1, keepdims=True)
    i1 = jnp.min(jnp.where(probs == m1, ids, E), axis=-1, keepdims=True)
    masked = jnp.where(ids == i1, -1.0, probs)
    m2 = jnp.max(masked, axis=-1, keepdims=True)
    i2 = jnp.min(jnp.where(masked == m2, ids, E), axis=-1, keepdims=True)
    denom = m1 + m2
    w_ref[...] = jnp.concatenate([m1 / denom, m2 / denom], axis=1)

    e_all = jnp.concatenate([i1, i2], axis=0)
    onehot = (e_all == lax.broadcasted_iota(jnp.int32, (P, E), 1)
              ).astype(jnp.float32)
    c = onehot
    d = 1
    while d < P:
        c = c + jnp.concatenate(
            [jnp.zeros((d, E), jnp.float32), c[:-d, :]], axis=0)
        d *= 2
    counts = c[P - 1:P, :]
    tiles = jnp.floor((counts + (TILE - 1)) * (1.0 / TILE))
    incl = tiles
    d = 1
    while d < E:
        incl = incl + jnp.concatenate(
            [jnp.zeros((1, d), jnp.float32), incl[:, :-d]], axis=1)
        d *= 2
    tb_excl = incl - tiles
    base = tb_excl * TILE
    slot = jnp.sum((base + c - 1.0) * onehot, axis=1, keepdims=True)
    inv_ref[...] = slot.astype(jnp.int32)

    jj = lax.broadcasted_iota(jnp.int32, (NT, E), 0).astype(jnp.float32)
    g = jnp.sum((incl <= jj).astype(jnp.float32), axis=1, keepdims=True)
    gidx_ref[...] = jnp.minimum(g, float(E - 1)).astype(jnp.int32)
    ntot_ref[...] = incl[:, E - 1:E].astype(jnp.int32)


def _gmm_outer(gidx_ref, ntot_ref, xs_hbm, w1_hbm, v1_hbm, w2_hbm, ys_hbm):
    ntiles = ntot_ref[0]

    def inner(xs_ref, w1_ref, v1_ref, w2_ref, ys_ref):
        xt = xs_ref[...]
        a = jnp.dot(xt, w1_ref[0], preferred_element_type=jnp.float32)
        bb = jnp.dot(xt, v1_ref[0], preferred_element_type=jnp.float32)
        h = a * jax.nn.sigmoid(a) * bb
        ys_ref[...] = jnp.dot(h, w2_ref[0], preferred_element_type=jnp.float32)

    wspec = lambda: pl.BlockSpec(
        (1, H, F), lambda j: (gidx_ref[j], 0, 0),
        pipeline_mode=pl.Buffered(buffer_count=2, use_lookahead=True))
    w2spec = pl.BlockSpec(
        (1, F, H), lambda j: (gidx_ref[j], 0, 0),
        pipeline_mode=pl.Buffered(buffer_count=2, use_lookahead=True))
    pltpu.emit_pipeline(
        inner,
        grid=(ntiles,),
        in_specs=[
            pl.BlockSpec((TILE, H), lambda j: (j, 0)),
            wspec(), wspec(), w2spec,
        ],
        out_specs=[pl.BlockSpec((TILE, H), lambda j: (j, 0))],
    )(xs_hbm, w1_hbm, v1_hbm, w2_hbm, ys_hbm)


def _combine_body(w_ref, y0_ref, y1_ref, out_ref):
    w0 = w_ref[:, 0:1]
    w1 = w_ref[:, 1:2]
    out_ref[...] = w0 * y0_ref[...] + w1 * y1_ref[...]


def _dispatch_x_body(x_hbm, inv_hbm, xs_hbm, idx_v, rows_v, sem):
    wid = lax.axis_index("s") * NC + lax.axis_index("c")
    pltpu.sync_copy(inv_hbm.at[pl.ds(wid * NCH, NCH)], idx_v)
    for c in range(NCH):
        t0 = lax.rem(wid * PPW + c * CH, S)
        pltpu.sync_copy(x_hbm.at[pl.ds(t0, CH)], rows_v)
        pltpu.async_copy(rows_v, xs_hbm.at[idx_v.at[c]], sem).wait()


def _gather_y_body(ys_hbm, inv_hbm, ysg_hbm, idx_v, rows_v, sem):
    wid = lax.axis_index("s") * NC + lax.axis_index("c")
    pltpu.sync_copy(inv_hbm.at[pl.ds(wid * NCH, NCH)], idx_v)
    for c in range(NCH):
        pltpu.async_copy(ys_hbm.at[idx_v.at[c]], rows_v, sem).wait()
        pltpu.sync_copy(rows_v, ysg_hbm.at[pl.ds(wid * PPW + c * CH, CH)])


def _sc_call(body, out_rows):
    mesh = plsc.VectorSubcoreMesh(
        core_axis_name="c", subcore_axis_name="s", num_cores=NC)
    return pl.kernel(
        body,
        mesh=mesh,
        out_type=jax.ShapeDtypeStruct((out_rows, H), jnp.float32),
        scratch_types=[
            pltpu.VMEM((NCH, CH), jnp.int32),
            pltpu.VMEM((CH, H), jnp.float32),
            pltpu.SemaphoreType.DMA,
        ],
    )


@jax.jit
def kernel(x, Wr, W1, V1, W2):
    b, s, hd = x.shape
    xf = x.reshape(s, hd)

    inv, wcomb, gidx, ntot = pl.pallas_call(
        _router_body,
        out_shape=(
            jax.ShapeDtypeStruct((P, 1), jnp.int32),
            jax.ShapeDtypeStruct((S, K), jnp.float32),
            jax.ShapeDtypeStruct((NT, 1), jnp.int32),
            jax.ShapeDtypeStruct((1, 1), jnp.int32),
        ),
    )(xf, Wr)

    inv_chunks = inv.reshape(NW * NCH, CH)

    xs = _sc_call(_dispatch_x_body, PAD)(xf, inv_chunks)

    ys = pl.pallas_call(
        _gmm_outer,
        grid_spec=pltpu.PrefetchScalarGridSpec(
            num_scalar_prefetch=2,
            grid=(1,),
            in_specs=[
                pl.BlockSpec(memory_space=pltpu.MemorySpace.HBM),
                pl.BlockSpec(memory_space=pltpu.MemorySpace.HBM),
                pl.BlockSpec(memory_space=pltpu.MemorySpace.HBM),
                pl.BlockSpec(memory_space=pltpu.MemorySpace.HBM),
            ],
            out_specs=pl.BlockSpec(memory_space=pltpu.MemorySpace.HBM),
        ),
        out_shape=jax.ShapeDtypeStruct((PAD, H), jnp.float32),
        compiler_params=pltpu.CompilerParams(
            vmem_limit_bytes=100 * 1024 * 1024),
    )(gidx.reshape(NT), ntot.reshape(1), xs, W1, V1, W2)

    ysg = _sc_call(_gather_y_body, P)(ys, inv_chunks)

    out = pl.pallas_call(
        _combine_body,
        grid=(1,),
        in_specs=[
            pl.BlockSpec((S, K), lambda i: (0, 0)),
            pl.BlockSpec((S, H), lambda i: (0, 0)),
            pl.BlockSpec((S, H), lambda i: (1, 0)),
        ],
        out_specs=pl.BlockSpec((S, H), lambda i: (0, 0)),
        out_shape=jax.ShapeDtypeStruct((S, H), jnp.float32),
    )(wcomb, ysg, ysg)

    return out.reshape(b, s, hd)


# speedup vs baseline: 1.8883x; 1.1003x over previous
import functools

import jax
import jax.numpy as jnp
from jax import lax
from jax.experimental import pallas as pl
from jax.experimental.pallas import tpu as pltpu
from jax.experimental.pallas import tpu_sc as plsc

H = 1024
F = 2048
E = 16
S = 2048
K = 2
P = K * S
TILE = 256
NT = 32
PAD = NT * TILE

NC, NS = 2, 16
NW = NC * NS
PPW = P // NW
CH = 32
NCH = PPW // CH


def _router_body(x_ref, wr_ref, inv_ref, w_ref, gidx_ref, ntot_ref):
    x = x_ref[...]
    logits = jnp.dot(x, wr_ref[...], preferred_element_type=jnp.float32)
    m = jnp.max(logits, axis=-1, keepdims=True)
    ex = jnp.exp(logits - m)
    probs = ex / jnp.sum(ex, axis=-1, keepdims=True)
    ids = lax.broadcasted_iota(jnp.int32, probs.shape, 1)
    m1 = jnp.max(probs, axis=-1, keepdims=True)
    i1 = jnp.min(jnp.where(probs == m1, ids, E), axis=-1, keepdims=True)
    masked = jnp.where(ids == i1, -1.0, probs)
    m2 = jnp.max(masked, axis=-1, keepdims=True)
    i2 = jnp.min(jnp.where(masked == m2, ids, E), axis=-1, keepdims=True)
    denom = m1 + m2
    w_ref[...] = jnp.concatenate([m1 / denom, m2 / denom], axis=1)

    e_all = jnp.concatenate([i1, i2], axis=0)
    onehot = (e_all == lax.broadcasted_iota(jnp.int32, (P, E), 1)
              ).astype(jnp.float32)
    c = onehot
    d = 1
    while d < P:
        c = c + jnp.concatenate(
            [jnp.zeros((d, E), jnp.float32), c[:-d, :]], axis=0)
        d *= 2
    counts = c[P - 1:P, :]
    tiles = jnp.floor((counts + (TILE - 1)) * (1.0 / TILE))
    incl = tiles
    d = 1
    while d < E:
        incl = incl + jnp.concatenate(
            [jnp.zeros((1, d), jnp.float32), incl[:, :-d]], axis=1)
        d *= 2
    tb_excl = incl - tiles
    base = tb_excl * TILE
    slot = jnp.sum((base + c - 1.0) * onehot, axis=1, keepdims=True)
    inv_ref[...] = slot.astype(jnp.int32)

    jj = lax.broadcasted_iota(jnp.int32, (NT, E), 0).astype(jnp.float32)
    g = jnp.sum((incl <= jj).astype(jnp.float32), axis=1, keepdims=True)
    gidx_ref[...] = jnp.minimum(g, float(E - 1)).astype(jnp.int32)
    ntot_ref[...] = incl[:, E - 1:E].astype(jnp.int32)


def _gmm_outer(gidx_ref, ntot_ref, xs_hbm, w1_hbm, v1_hbm, w2_hbm, ys_hbm):
    ntiles = ntot_ref[0]

    def inner(xs_ref, w1_ref, v1_ref, w2_ref, ys_ref):
        xt = xs_ref[...]
        a = jnp.dot(xt, w1_ref[0], preferred_element_type=jnp.float32)
        bb = jnp.dot(xt, v1_ref[0], preferred_element_type=jnp.float32)
        h = a * jax.nn.sigmoid(a) * bb
        ys_ref[...] = jnp.dot(h, w2_ref[0], preferred_element_type=jnp.float32)

    wspec = lambda: pl.BlockSpec(
        (1, H, F), lambda j: (gidx_ref[j], 0, 0),
        pipeline_mode=pl.Buffered(buffer_count=2, use_lookahead=True))
    w2spec = pl.BlockSpec(
        (1, F, H), lambda j: (gidx_ref[j], 0, 0),
        pipeline_mode=pl.Buffered(buffer_count=2, use_lookahead=True))
    pltpu.emit_pipeline(
        inner,
        grid=(ntiles,),
        in_specs=[
            pl.BlockSpec((TILE, H), lambda j: (j, 0)),
            wspec(), wspec(), w2spec,
        ],
        out_specs=[pl.BlockSpec((TILE, H), lambda j: (j, 0))],
    )(xs_hbm, w1_hbm, v1_hbm, w2_hbm, ys_hbm)


def _combine_body(w_ref, y0_ref, y1_ref, out_ref):
    w0 = w_ref[:, 0:1]
    w1 = w_ref[:, 1:2]
    out_ref[...] = w0 * y0_ref[...] + w1 * y1_ref[...]


def _dispatch_x_body(x_hbm, inv_hbm, xs_hbm, idx_v, rows0, rows1,
                     isem0, isem1, osem0, osem1):
    wid = lax.axis_index("s") * NC + lax.axis_index("c")
    pltpu.sync_copy(inv_hbm.at[pl.ds(wid * NCH, NCH)], idx_v)
    rows = (rows0, rows1)
    isem = (isem0, isem1)
    osem = (osem0, osem1)

    def start_in(c, b):
        t0 = lax.rem(wid * PPW + c * CH, S)
        pltpu.make_async_copy(x_hbm.at[pl.ds(t0, CH)], rows[b], isem[b]).start()

    start_in(0, 0)
    start_in(1, 1)
    for c in range(NCH):
        b = c % 2
        pltpu.make_async_copy(
            x_hbm.at[pl.ds(0, CH)], rows[b], isem[b]).wait()
        scat = pltpu.make_async_copy(rows[b], xs_hbm.at[idx_v.at[c]], osem[b])
        scat.start()
        if c + 2 < NCH:
            scat.wait()
            start_in(c + 2, b)
        else:
            scat.wait()


def _gather_y_body(ys_hbm, inv_hbm, ysg_hbm, idx_v, rows0, rows1,
                   isem0, isem1, osem0, osem1):
    wid = lax.axis_index("s") * NC + lax.axis_index("c")
    pltpu.sync_copy(inv_hbm.at[pl.ds(wid * NCH, NCH)], idx_v)
    rows = (rows0, rows1)
    isem = (isem0, isem1)
    osem = (osem0, osem1)

    def start_gat(c, b):
        pltpu.make_async_copy(ys_hbm.at[idx_v.at[c]], rows[b], isem[b]).start()

    start_gat(0, 0)
    start_gat(1, 1)
    for c in range(NCH):
        b = c % 2
        pltpu.make_async_copy(
            ys_hbm.at[idx_v.at[c]], rows[b], isem[b]).wait()
        out = pltpu.make_async_copy(
            rows[b], ysg_hbm.at[pl.ds(wid * PPW + c * CH, CH)], osem[b])
        out.start()
        if c + 2 < NCH:
            out.wait()
            start_gat(c + 2, b)
        else:
            out.wait()


def _sc_call(body, out_rows):
    mesh = plsc.VectorSubcoreMesh(
        core_axis_name="c", subcore_axis_name="s", num_cores=NC)
    return pl.kernel(
        body,
        mesh=mesh,
        out_type=jax.ShapeDtypeStruct((out_rows, H), jnp.float32),
        scratch_types=[
            pltpu.VMEM((NCH, CH), jnp.int32),
            pltpu.VMEM((CH, H), jnp.float32),
            pltpu.VMEM((CH, H), jnp.float32),
            pltpu.SemaphoreType.DMA,
            pltpu.SemaphoreType.DMA,
            pltpu.SemaphoreType.DMA,
            pltpu.SemaphoreType.DMA,
        ],
    )


@jax.jit
def kernel(x, Wr, W1, V1, W2):
    b, s, hd = x.shape
    xf = x.reshape(s, hd)

    inv, wcomb, gidx, ntot = pl.pallas_call(
        _router_body,
        out_shape=(
            jax.ShapeDtypeStruct((P, 1), jnp.int32),
            jax.ShapeDtypeStruct((S, K), jnp.float32),
            jax.ShapeDtypeStruct((NT, 1), jnp.int32),
            jax.ShapeDtypeStruct((1, 1), jnp.int32),
        ),
    )(xf, Wr)

    inv_chunks = inv.reshape(NW * NCH, CH)

    xs = _sc_call(_dispatch_x_body, PAD)(xf, inv_chunks)

    ys = pl.pallas_call(
        _gmm_outer,
        grid_spec=pltpu.PrefetchScalarGridSpec(
            num_scalar_prefetch=2,
            grid=(1,),
            in_specs=[
                pl.BlockSpec(memory_space=pltpu.MemorySpace.HBM),
                pl.BlockSpec(memory_space=pltpu.MemorySpace.HBM),
                pl.BlockSpec(memory_space=pltpu.MemorySpace.HBM),
                pl.BlockSpec(memory_space=pltpu.MemorySpace.HBM),
            ],
            out_specs=pl.BlockSpec(memory_space=pltpu.MemorySpace.HBM),
        ),
        out_shape=jax.ShapeDtypeStruct((PAD, H), jnp.float32),
        compiler_params=pltpu.CompilerParams(
            vmem_limit_bytes=100 * 1024 * 1024),
    )(gidx.reshape(NT), ntot.reshape(1), xs, W1, V1, W2)

    ysg = _sc_call(_gather_y_body, P)(ys, inv_chunks)

    out = pl.pallas_call(
        _combine_body,
        grid=(1,),
        in_specs=[
            pl.BlockSpec((S, K), lambda i: (0, 0)),
            pl.BlockSpec((S, H), lambda i: (0, 0)),
            pl.BlockSpec((S, H), lambda i: (1, 0)),
        ],
        out_specs=pl.BlockSpec((S, H), lambda i: (0, 0)),
        out_shape=jax.ShapeDtypeStruct((S, H), jnp.float32),
    )(wcomb, ysg, ysg)

    return out.reshape(b, s, hd)
